# Initial kernel scaffold; baseline (speedup 1.0000x reference)
import functools

import jax
import jax.numpy as jnp
from jax import lax
from jax.experimental import pallas as pl
from jax.experimental.pallas import tpu as pltpu
from jax.experimental.pallas import tpu_sc as plsc

N = 50000
E = 800000
D = 32
H = 4
DH = 8

NC = 2
NS = 16
L = 16
NW = NC * NS
EPW = E // NW
CH = 1000
NCHUNK = EPW // CH
RPT = N // NS

BN = 5000
BE = 8000


def _node_proj_body(x_ref, w_ref, b_ref, asrc_ref, adst_ref, h_ref, ts_ref, td_ref):
    h = jax.nn.gelu(jnp.dot(x_ref[...], w_ref[...],
                            preferred_element_type=jnp.float32) + b_ref[...])
    h_ref[...] = h
    ts_ref[...] = jnp.dot(h, asrc_ref[...], preferred_element_type=jnp.float32)
    td_ref[...] = jnp.dot(h, adst_ref[...], preferred_element_type=jnp.float32)


def _edge_proj_body(ea_ref, w_ref, b_ref, aedge_ref, pe_ref):
    e = jax.nn.gelu(jnp.dot(ea_ref[...], w_ref[...],
                            preferred_element_type=jnp.float32) + b_ref[...])
    pe_ref[...] = jnp.dot(e, aedge_ref[...], preferred_element_type=jnp.float32)


def _out_proj_body(u_ref, d_ref, h_ref, rmat_ref, w_ref, b_ref, o_ref):
    u = u_ref[0] + u_ref[1]
    dn = d_ref[0] + d_ref[1]
    rep = jnp.dot(dn, rmat_ref[...], preferred_element_type=jnp.float32)
    agg = u / (rep + 1e-20)
    o_ref[...] = jax.nn.gelu(jnp.dot(agg, w_ref[...],
                                     preferred_element_type=jnp.float32)
                             + b_ref[...]) + h_ref[...]


_SC_MESH = plsc.VectorSubcoreMesh(core_axis_name="c", subcore_axis_name="s")


@functools.partial(
    pl.kernel,
    out_type=(
        jax.ShapeDtypeStruct((NC * N, D), jnp.float32),
        jax.ShapeDtypeStruct((NC * N, H), jnp.float32),
    ),
    mesh=_SC_MESH,
    scratch_types=(
        pltpu.VMEM_SHARED((N, D), jnp.float32),
        pltpu.VMEM_SHARED((N, H), jnp.float32),
        pltpu.VMEM((CH,), jnp.int32),
        pltpu.VMEM((CH,), jnp.int32),
        pltpu.VMEM((CH, H), jnp.float32),
        pltpu.VMEM((CH, H), jnp.float32),
        pltpu.VMEM((CH, H), jnp.float32),
        pltpu.VMEM((CH, H), jnp.float32),
        pltpu.VMEM((CH, D), jnp.float32),
        pltpu.VMEM((CH, D), jnp.float32),
        pltpu.SemaphoreType.DMA,
    ),
)
def _sc_edge_pass(src_hbm, dst_hbm, ts_hbm, td_hbm, pe_hbm, h_hbm,
                  z32_hbm, z4_hbm, u_out, d_out,
                  u_sh, d_sh, srcv, dstv, sg, dg, pev, evv, hsv, msgv, sem):
    cid = lax.axis_index("c")
    sid = lax.axis_index("s")
    wid = cid * NS + sid
    row0 = sid * RPT

    pltpu.sync_copy(z32_hbm.at[pl.ds(row0, RPT)], u_sh.at[pl.ds(row0, RPT)])
    pltpu.sync_copy(z4_hbm.at[pl.ds(row0, RPT)], d_sh.at[pl.ds(row0, RPT)])
    plsc.subcore_barrier()

    iota = lax.iota(jnp.int32, L)
    row4 = iota // H
    col4 = lax.rem(iota, H)
    bidx = iota // DH
    ebase = wid * EPW

    @pl.loop(0, NCHUNK)
    def _chunk(j):
        off = ebase + j * CH
        pltpu.sync_copy(src_hbm.at[pl.ds(off, CH)], srcv)
        pltpu.sync_copy(dst_hbm.at[pl.ds(off, CH)], dstv)
        pltpu.sync_copy(pe_hbm.at[pl.ds(off, CH)], pev)
        c1 = pltpu.async_copy(ts_hbm.at[srcv], sg, sem)
        c2 = pltpu.async_copy(td_hbm.at[dstv], dg, sem)
        c3 = pltpu.async_copy(h_hbm.at[srcv], hsv, sem)
        c1.wait()
        c2.wait()
        c3.wait()

        @pl.loop(0, CH * H // L)
        def _ev(i):
            rows = row4 + i * (L // H)
            a = plsc.load_gather(sg, [rows, col4])
            b = plsc.load_gather(dg, [rows, col4])
            c = plsc.load_gather(pev, [rows, col4])
            s = a + b + c
            s = jnp.maximum(s, 0.2 * s)
            plsc.store_scatter(evv, [rows, col4], jnp.exp(s))

        @pl.loop(0, CH)
        def _msg(r):
            rows = jnp.full((L,), 0, jnp.int32) + r
            for half in range(2):
                cols = iota + (half * L)
                hvals = plsc.load_gather(hsv, [rows, cols])
                evb = plsc.load_gather(evv, [rows, bidx + (half * 2)])
                plsc.store_scatter(msgv, [rows, cols], hvals * evb)

        pltpu.sync_copy(evv, d_sh.at[dstv], add=True)
        pltpu.sync_copy(msgv, u_sh.at[dstv], add=True)

    plsc.subcore_barrier()
    roff = cid * N + row0
    pltpu.sync_copy(u_sh.at[pl.ds(row0, RPT)], u_out.at[pl.ds(roff, RPT)])
    pltpu.sync_copy(d_sh.at[pl.ds(row0, RPT)], d_out.at[pl.ds(roff, RPT)])


def kernel(x, edge_index, edge_attr, W_in, b_in, W_edge, b_edge,
           a_src, a_dst, a_edge, W_out, b_out):
    src = edge_index[0]
    dst = edge_index[1]
    f_in = x.shape[1]
    f_edge = edge_attr.shape[1]

    mask = (jnp.arange(D)[:, None] // DH) == jnp.arange(H)[None, :]
    A_src = jnp.where(mask, a_src.reshape(D)[:, None], 0.0).astype(jnp.float32)
    A_dst = jnp.where(mask, a_dst.reshape(D)[:, None], 0.0).astype(jnp.float32)
    A_edge = jnp.where(mask, a_edge.reshape(D)[:, None], 0.0).astype(jnp.float32)
    Rmat = mask.T.astype(jnp.float32)

    h, ts, td = pl.pallas_call(
        _node_proj_body,
        grid=(N // BN,),
        in_specs=[
            pl.BlockSpec((BN, f_in), lambda i: (i, 0)),
            pl.BlockSpec((f_in, D), lambda i: (0, 0)),
            pl.BlockSpec((1, D), lambda i: (0, 0)),
            pl.BlockSpec((D, H), lambda i: (0, 0)),
            pl.BlockSpec((D, H), lambda i: (0, 0)),
        ],
        out_specs=[
            pl.BlockSpec((BN, D), lambda i: (i, 0)),
            pl.BlockSpec((BN, H), lambda i: (i, 0)),
            pl.BlockSpec((BN, H), lambda i: (i, 0)),
        ],
        out_shape=[
            jax.ShapeDtypeStruct((N, D), jnp.float32),
            jax.ShapeDtypeStruct((N, H), jnp.float32),
            jax.ShapeDtypeStruct((N, H), jnp.float32),
        ],
    )(x, W_in, b_in.reshape(1, D), A_src, A_dst)

    pe = pl.pallas_call(
        _edge_proj_body,
        grid=(E // BE,),
        in_specs=[
            pl.BlockSpec((BE, f_edge), lambda i: (i, 0)),
            pl.BlockSpec((f_edge, D), lambda i: (0, 0)),
            pl.BlockSpec((1, D), lambda i: (0, 0)),
            pl.BlockSpec((D, H), lambda i: (0, 0)),
        ],
        out_specs=pl.BlockSpec((BE, H), lambda i: (i, 0)),
        out_shape=jax.ShapeDtypeStruct((E, H), jnp.float32),
    )(edge_attr, W_edge, b_edge.reshape(1, D), A_edge)

    z32 = jnp.zeros((N, D), jnp.float32)
    z4 = jnp.zeros((N, H), jnp.float32)
    u2, d2 = _sc_edge_pass(src, dst, ts, td, pe, h, z32, z4)
    u2 = u2.reshape(NC, N, D)
    d2 = d2.reshape(NC, N, H)

    out = pl.pallas_call(
        _out_proj_body,
        grid=(N // BN,),
        in_specs=[
            pl.BlockSpec((NC, BN, D), lambda i: (0, i, 0)),
            pl.BlockSpec((NC, BN, H), lambda i: (0, i, 0)),
            pl.BlockSpec((BN, D), lambda i: (i, 0)),
            pl.BlockSpec((H, D), lambda i: (0, 0)),
            pl.BlockSpec((D, D), lambda i: (0, 0)),
            pl.BlockSpec((1, D), lambda i: (0, 0)),
        ],
        out_specs=pl.BlockSpec((BN, D), lambda i: (i, 0)),
        out_shape=jax.ShapeDtypeStruct((N, D), jnp.float32),
    )(u2, d2, h, Rmat, W_out, b_out.reshape(1, D))
    return out


# speedup vs baseline: 48.1250x; 48.1250x over previous
import functools

import jax
import jax.numpy as jnp
from jax import lax
from jax.experimental import pallas as pl
from jax.experimental.pallas import tpu as pltpu
from jax.experimental.pallas import tpu_sc as plsc

N = 50000
E = 800000
D = 32
H = 4
DH = 8

NC = 2
NS = 16
L = 16
NW = NC * NS
EPW = E // NW
CH = 200
NCHUNK = EPW // CH
NPAD = 50048
RPT = NPAD // NS

BN = 5000
BE = 8000


def _node_proj_body(x_ref, w_ref, b_ref, asd_ref, h_ref, t_ref):
    h = jax.nn.gelu(jnp.dot(x_ref[...], w_ref[...],
                            preferred_element_type=jnp.float32) + b_ref[...])
    h_ref[...] = h
    t_ref[...] = jnp.dot(h, asd_ref[...], preferred_element_type=jnp.float32)


def _edge_proj_body(ea_ref, w_ref, b_ref, aedge_ref, pe_ref):
    e = jax.nn.gelu(jnp.dot(ea_ref[...], w_ref[...],
                            preferred_element_type=jnp.float32) + b_ref[...])
    pe_ref[...] = jnp.dot(e, aedge_ref[...], preferred_element_type=jnp.float32)


def _out_proj_body(u_ref, d_ref, h_ref, rmat_ref, w_ref, b_ref, o_ref):
    u = u_ref[0] + u_ref[1]
    dn = d_ref[0] + d_ref[1]
    rep = jnp.dot(dn, rmat_ref[...], preferred_element_type=jnp.float32)
    agg = u / (rep + 1e-20)
    o_ref[...] = jax.nn.gelu(jnp.dot(agg, w_ref[...],
                                     preferred_element_type=jnp.float32)
                             + b_ref[...]) + h_ref[...]


_SC_MESH = plsc.VectorSubcoreMesh(core_axis_name="c", subcore_axis_name="s")
_SC_PARAMS = pltpu.CompilerParams(use_tc_tiling_on_sc=False,
                                  needs_layout_passes=False)
CH1 = 1000
CH2 = 200


@functools.partial(
    pl.kernel,
    out_type=(
        jax.ShapeDtypeStruct((E, DH), jnp.float32),
        jax.ShapeDtypeStruct((NC * NPAD, DH), jnp.float32),
    ),
    mesh=_SC_MESH,
    compiler_params=_SC_PARAMS,
    scratch_types=(
        pltpu.VMEM_SHARED((NPAD, DH), jnp.float32),
        pltpu.VMEM((CH1,), jnp.int32),
        pltpu.VMEM((CH1,), jnp.int32),
        pltpu.VMEM((CH1, DH), jnp.float32),
        pltpu.VMEM((CH1, DH), jnp.float32),
        pltpu.VMEM((CH1, H), jnp.float32),
        pltpu.VMEM((CH1, DH), jnp.float32),
        pltpu.SemaphoreType.DMA,
    ),
)
def _sc_logit_pass(src_hbm, dst_hbm, t_hbm, pe_hbm, z8_hbm,
                   ev_out, d_out, d_sh, srcv, dstv, sg, dg, pev, evv, sem):
    cid = lax.axis_index("c")
    sid = lax.axis_index("s")
    wid = cid * NS + sid
    row0 = sid * RPT
    pltpu.sync_copy(z8_hbm.at[pl.ds(row0, RPT)], d_sh.at[pl.ds(row0, RPT)])
    pltpu.sync_copy(z8_hbm.at[pl.ds(0, CH1)], evv)
    plsc.subcore_barrier()

    iota = lax.iota(jnp.int32, L)
    row4 = iota // H
    col4 = lax.rem(iota, H)
    ebase = wid * EPW

    @pl.loop(0, EPW // CH1)
    def _chunk(j):
        off = ebase + j * CH1
        pltpu.sync_copy(src_hbm.at[pl.ds(off, CH1)], srcv)
        pltpu.sync_copy(dst_hbm.at[pl.ds(off, CH1)], dstv)
        pltpu.sync_copy(pe_hbm.at[pl.ds(off, CH1)], pev)
        c1 = pltpu.async_copy(t_hbm.at[srcv], sg, sem)
        c2 = pltpu.async_copy(t_hbm.at[dstv], dg, sem)
        c1.wait()
        c2.wait()

        @pl.loop(0, CH1 * H // L)
        def _ev(i):
            rows = row4 + i * (L // H)
            a = plsc.load_gather(sg, [rows, col4])
            b = plsc.load_gather(dg, [rows, col4 + H])
            c = plsc.load_gather(pev, [rows, col4])
            s = a + b + c
            s = jnp.maximum(s, 0.2 * s)
            plsc.store_scatter(evv, [rows, col4], jnp.exp(s))

        pltpu.sync_copy(evv, d_sh.at[dstv], add=True)
        pltpu.sync_copy(evv, ev_out.at[pl.ds(off, CH1)])

    plsc.subcore_barrier()
    roff = cid * NPAD + row0
    pltpu.sync_copy(d_sh.at[pl.ds(row0, RPT)], d_out.at[pl.ds(roff, RPT)])


@functools.partial(
    pl.kernel,
    out_type=jax.ShapeDtypeStruct((NC * NPAD, D), jnp.float32),
    mesh=_SC_MESH,
    compiler_params=_SC_PARAMS,
    scratch_types=(
        pltpu.VMEM_SHARED((NPAD, D), jnp.float32),
        pltpu.VMEM((CH2,), jnp.int32),
        pltpu.VMEM((CH2,), jnp.int32),
        pltpu.VMEM((CH2, DH), jnp.float32),
        pltpu.VMEM((CH2, D), jnp.float32),
        pltpu.SemaphoreType.DMA,
    ),
)
def _sc_agg_pass(src_hbm, dst_hbm, ev_hbm, h_hbm, z32_hbm,
                 u_out, u_sh, srcv, dstv, evv, hsv, sem):
    cid = lax.axis_index("c")
    sid = lax.axis_index("s")
    wid = cid * NS + sid
    row0 = sid * RPT
    pltpu.sync_copy(z32_hbm.at[pl.ds(row0, RPT)], u_sh.at[pl.ds(row0, RPT)])
    plsc.subcore_barrier()

    iota = lax.iota(jnp.int32, L)
    bidx = iota // DH
    ebase = wid * EPW

    @pl.loop(0, EPW // CH2)
    def _chunk(j):
        off = ebase + j * CH2
        pltpu.sync_copy(src_hbm.at[pl.ds(off, CH2)], srcv)
        pltpu.sync_copy(dst_hbm.at[pl.ds(off, CH2)], dstv)
        pltpu.sync_copy(ev_hbm.at[pl.ds(off, CH2)], evv)
        pltpu.async_copy(h_hbm.at[srcv], hsv, sem).wait()

        @pl.loop(0, CH2)
        def _msg(r):
            rows = jnp.full((L,), 0, jnp.int32) + r
            for half in range(2):
                cols = iota + (half * L)
                hvals = plsc.load_gather(hsv, [rows, cols])
                evb = plsc.load_gather(evv, [rows, bidx + (half * 2)])
                plsc.store_scatter(hsv, [rows, cols], hvals * evb)

        pltpu.sync_copy(hsv, u_sh.at[dstv], add=True)

    plsc.subcore_barrier()
    roff = cid * NPAD + row0
    pltpu.sync_copy(u_sh.at[pl.ds(row0, RPT)], u_out.at[pl.ds(roff, RPT)])


def kernel(x, edge_index, edge_attr, W_in, b_in, W_edge, b_edge,
           a_src, a_dst, a_edge, W_out, b_out):
    src = edge_index[0]
    dst = edge_index[1]
    f_in = x.shape[1]
    f_edge = edge_attr.shape[1]

    mask = (jnp.arange(D)[:, None] // DH) == jnp.arange(H)[None, :]
    A_src = jnp.where(mask, a_src.reshape(D)[:, None], 0.0).astype(jnp.float32)
    A_dst = jnp.where(mask, a_dst.reshape(D)[:, None], 0.0).astype(jnp.float32)
    A_sd = jnp.concatenate([A_src, A_dst], axis=1)
    A_edge = jnp.where(mask, a_edge.reshape(D)[:, None], 0.0).astype(jnp.float32)
    Rmat = jnp.concatenate([mask.T.astype(jnp.float32),
                            jnp.zeros((H, D), jnp.float32)], axis=0)

    h, t = pl.pallas_call(
        _node_proj_body,
        grid=(N // BN,),
        in_specs=[
            pl.BlockSpec((BN, f_in), lambda i: (i, 0)),
            pl.BlockSpec((f_in, D), lambda i: (0, 0)),
            pl.BlockSpec((1, D), lambda i: (0, 0)),
            pl.BlockSpec((D, DH), lambda i: (0, 0)),
        ],
        out_specs=[
            pl.BlockSpec((BN, D), lambda i: (i, 0)),
            pl.BlockSpec((BN, DH), lambda i: (i, 0)),
        ],
        out_shape=[
            jax.ShapeDtypeStruct((N, D), jnp.float32),
            jax.ShapeDtypeStruct((N, DH), jnp.float32),
        ],
    )(x, W_in, b_in.reshape(1, D), A_sd)

    pe = pl.pallas_call(
        _edge_proj_body,
        grid=(E // BE,),
        in_specs=[
            pl.BlockSpec((BE, f_edge), lambda i: (i, 0)),
            pl.BlockSpec((f_edge, D), lambda i: (0, 0)),
            pl.BlockSpec((1, D), lambda i: (0, 0)),
            pl.BlockSpec((D, H), lambda i: (0, 0)),
        ],
        out_specs=pl.BlockSpec((BE, H), lambda i: (i, 0)),
        out_shape=jax.ShapeDtypeStruct((E, H), jnp.float32),
    )(edge_attr, W_edge, b_edge.reshape(1, D), A_edge)

    z8 = jnp.zeros((NPAD, DH), jnp.float32)
    z32 = jnp.zeros((NPAD, D), jnp.float32)
    ev8, d2 = _sc_logit_pass(src, dst, t, pe, z8)
    u2 = _sc_agg_pass(src, dst, ev8, h, z32)
    u2 = u2.reshape(NC, NPAD, D)[:, :N]
    d2 = d2.reshape(NC, NPAD, DH)[:, :N]

    out = pl.pallas_call(
        _out_proj_body,
        grid=(N // BN,),
        in_specs=[
            pl.BlockSpec((NC, BN, D), lambda i: (0, i, 0)),
            pl.BlockSpec((NC, BN, DH), lambda i: (0, i, 0)),
            pl.BlockSpec((BN, D), lambda i: (i, 0)),
            pl.BlockSpec((DH, D), lambda i: (0, 0)),
            pl.BlockSpec((D, D), lambda i: (0, 0)),
            pl.BlockSpec((1, D), lambda i: (0, 0)),
        ],
        out_specs=pl.BlockSpec((BN, D), lambda i: (i, 0)),
        out_shape=jax.ShapeDtypeStruct((N, D), jnp.float32),
    )(u2, d2, h, Rmat, W_out, b_out.reshape(1, D))
    return out


# speedup vs baseline: 55.6108x; 1.1555x over previous
import functools

import jax
import jax.numpy as jnp
from jax import lax
from jax.experimental import pallas as pl
from jax.experimental.pallas import tpu as pltpu
from jax.experimental.pallas import tpu_sc as plsc

N = 50000
E = 800000
D = 32
H = 4
DH = 8

NC = 2
NS = 16
L = 16
NW = NC * NS
EPW = E // NW
CH = 200
NCHUNK = EPW // CH
NPAD = 50048
RPT = NPAD // NS

BN = 5000
BE = 8000


def _node_proj_body(x_ref, w_ref, b_ref, asd_ref, h_ref, t_ref):
    h = jax.nn.gelu(jnp.dot(x_ref[...], w_ref[...],
                            preferred_element_type=jnp.float32) + b_ref[...])
    h_ref[...] = h
    t_ref[...] = jnp.dot(h, asd_ref[...], preferred_element_type=jnp.float32)


def _edge_proj_body(ea_ref, w_ref, b_ref, aedge_ref, pe_ref):
    e = jax.nn.gelu(jnp.dot(ea_ref[...], w_ref[...],
                            preferred_element_type=jnp.float32) + b_ref[...])
    pe_ref[...] = jnp.dot(e, aedge_ref[...], preferred_element_type=jnp.float32)


def _out_proj_body(u_ref, d_ref, h_ref, rmat_ref, w_ref, b_ref, o_ref):
    u = u_ref[0] + u_ref[1]
    dn = d_ref[0] + d_ref[1]
    rep = jnp.dot(dn, rmat_ref[...], preferred_element_type=jnp.float32)
    agg = u / (rep + 1e-20)
    o_ref[...] = jax.nn.gelu(jnp.dot(agg, w_ref[...],
                                     preferred_element_type=jnp.float32)
                             + b_ref[...]) + h_ref[...]


_SC_MESH = plsc.VectorSubcoreMesh(core_axis_name="c", subcore_axis_name="s")
_SC_PARAMS = pltpu.CompilerParams(use_tc_tiling_on_sc=False,
                                  needs_layout_passes=False)
CH1 = 1000
CH2 = 720
NCH2 = EPW // CH2
REM2 = EPW - NCH2 * CH2


@functools.partial(
    pl.kernel,
    out_type=(
        jax.ShapeDtypeStruct((E, DH), jnp.float32),
        jax.ShapeDtypeStruct((NC, NPAD, DH), jnp.float32),
    ),
    mesh=_SC_MESH,
    compiler_params=_SC_PARAMS,
    scratch_types=(
        pltpu.VMEM_SHARED((NPAD, DH), jnp.float32),
        pltpu.VMEM((CH1,), jnp.int32),
        pltpu.VMEM((CH1,), jnp.int32),
        pltpu.VMEM((CH1, DH), jnp.float32),
        pltpu.VMEM((CH1, DH), jnp.float32),
        pltpu.VMEM((CH1, H), jnp.float32),
        pltpu.VMEM((CH1, DH), jnp.float32),
        pltpu.SemaphoreType.DMA,
    ),
)
def _sc_logit_pass(src_hbm, dst_hbm, t_hbm, pe_hbm, z8_hbm,
                   ev_out, d_out, d_sh, srcv, dstv, sg, dg, pev, evv, sem):
    cid = lax.axis_index("c")
    sid = lax.axis_index("s")
    wid = cid * NS + sid
    row0 = sid * RPT
    pltpu.sync_copy(z8_hbm.at[pl.ds(row0, RPT)], d_sh.at[pl.ds(row0, RPT)])
    pltpu.sync_copy(z8_hbm.at[pl.ds(0, CH1)], evv)
    plsc.subcore_barrier()

    iota = lax.iota(jnp.int32, L)
    row4 = iota // H
    col4 = lax.rem(iota, H)
    ebase = wid * EPW

    @pl.loop(0, EPW // CH1)
    def _chunk(j):
        off = ebase + j * CH1
        i1 = pltpu.async_copy(src_hbm.at[pl.ds(off, CH1)], srcv, sem)
        i2 = pltpu.async_copy(dst_hbm.at[pl.ds(off, CH1)], dstv, sem)
        i3 = pltpu.async_copy(pe_hbm.at[pl.ds(off, CH1)], pev, sem)
        i1.wait()
        c1 = pltpu.async_copy(t_hbm.at[srcv], sg, sem)
        i2.wait()
        c2 = pltpu.async_copy(t_hbm.at[dstv], dg, sem)
        i3.wait()
        c1.wait()
        c2.wait()

        @pl.loop(0, CH1 * H // L, unroll=4)
        def _ev(i):
            rows = row4 + i * (L // H)
            a = plsc.load_gather(sg, [rows, col4])
            b = plsc.load_gather(dg, [rows, col4 + H])
            c = plsc.load_gather(pev, [rows, col4])
            s = a + b + c
            s = jnp.maximum(s, 0.2 * s)
            plsc.store_scatter(evv, [rows, col4], jnp.exp(s))

        pltpu.sync_copy(evv, d_sh.at[dstv], add=True)
        pltpu.sync_copy(evv, ev_out.at[pl.ds(off, CH1)])

    plsc.subcore_barrier()
    pltpu.sync_copy(d_sh.at[pl.ds(row0, RPT)], d_out.at[cid, pl.ds(row0, RPT)])


@functools.partial(
    pl.kernel,
    out_type=jax.ShapeDtypeStruct((NC, NPAD, D), jnp.float32),
    mesh=_SC_MESH,
    compiler_params=_SC_PARAMS,
    scratch_types=(
        pltpu.VMEM_SHARED((NPAD, D), jnp.float32),
        pltpu.VMEM((CH2,), jnp.int32),
        pltpu.VMEM((CH2,), jnp.int32),
        pltpu.VMEM((CH2, DH), jnp.float32),
        pltpu.VMEM((CH2, D), jnp.float32),
        pltpu.SemaphoreType.DMA,
    ),
)
def _sc_agg_pass(src_hbm, dst_hbm, ev_hbm, h_hbm, z32_hbm,
                 u_out, u_sh, srcv, dstv, evv, hsv, sem):
    cid = lax.axis_index("c")
    sid = lax.axis_index("s")
    wid = cid * NS + sid
    row0 = sid * RPT
    pltpu.sync_copy(z32_hbm.at[pl.ds(row0, RPT)], u_sh.at[pl.ds(row0, RPT)])
    plsc.subcore_barrier()

    iota = lax.iota(jnp.int32, L)
    bidx = iota // DH
    ebase = wid * EPW

    def do_chunk(off, n_edges):
        i1 = pltpu.async_copy(src_hbm.at[pl.ds(off, n_edges)],
                              srcv.at[pl.ds(0, n_edges)], sem)
        i2 = pltpu.async_copy(dst_hbm.at[pl.ds(off, n_edges)],
                              dstv.at[pl.ds(0, n_edges)], sem)
        i3 = pltpu.async_copy(ev_hbm.at[pl.ds(off, n_edges)],
                              evv.at[pl.ds(0, n_edges)], sem)
        i1.wait()
        cg = pltpu.async_copy(h_hbm.at[srcv], hsv, sem)
        i2.wait()
        i3.wait()
        cg.wait()

        @pl.loop(0, CH2, unroll=8)
        def _msg(r):
            rows = jnp.full((L,), 0, jnp.int32) + r
            for half in range(2):
                cols = iota + (half * L)
                hvals = plsc.load_gather(hsv, [rows, cols])
                evb = plsc.load_gather(evv, [rows, bidx + (half * 2)])
                plsc.store_scatter(hsv, [rows, cols], hvals * evb)

        pltpu.sync_copy(hsv, u_sh.at[dstv], add=True)

    @pl.loop(0, NCH2)
    def _chunk(j):
        do_chunk(ebase + j * CH2, CH2)

    @pl.loop(REM2 // L, CH2 // L)
    def _fill(q):
        dstv[pl.ds(q * L, L)] = N + lax.rem(iota + q, jnp.int32(NPAD - N))
    do_chunk(ebase + NCH2 * CH2, REM2)

    plsc.subcore_barrier()
    pltpu.sync_copy(u_sh.at[pl.ds(row0, RPT)],
                    u_out.at[cid, pl.ds(row0, RPT)])


def kernel(x, edge_index, edge_attr, W_in, b_in, W_edge, b_edge,
           a_src, a_dst, a_edge, W_out, b_out):
    src = edge_index[0]
    dst = edge_index[1]
    f_in = x.shape[1]
    f_edge = edge_attr.shape[1]
    BN1 = NPAD // 16

    mask = (jnp.arange(D)[:, None] // DH) == jnp.arange(H)[None, :]
    A_src = jnp.where(mask, a_src.reshape(D)[:, None], 0.0).astype(jnp.float32)
    A_dst = jnp.where(mask, a_dst.reshape(D)[:, None], 0.0).astype(jnp.float32)
    A_sd = jnp.concatenate([A_src, A_dst], axis=1)
    A_edge = jnp.where(mask, a_edge.reshape(D)[:, None], 0.0).astype(jnp.float32)
    Rmat = jnp.concatenate([mask.T.astype(jnp.float32),
                            jnp.zeros((H, D), jnp.float32)], axis=0)

    x_pad = jnp.zeros((NPAD, f_in), jnp.float32).at[:N].set(x)

    h, t = pl.pallas_call(
        _node_proj_body,
        grid=(16,),
        in_specs=[
            pl.BlockSpec((BN1, f_in), lambda i: (i, 0)),
            pl.BlockSpec((f_in, D), lambda i: (0, 0)),
            pl.BlockSpec((1, D), lambda i: (0, 0)),
            pl.BlockSpec((D, DH), lambda i: (0, 0)),
        ],
        out_specs=[
            pl.BlockSpec((BN1, D), lambda i: (i, 0)),
            pl.BlockSpec((BN1, DH), lambda i: (i, 0)),
        ],
        out_shape=[
            jax.ShapeDtypeStruct((NPAD, D), jnp.float32),
            jax.ShapeDtypeStruct((NPAD, DH), jnp.float32),
        ],
    )(x_pad, W_in, b_in.reshape(1, D), A_sd)

    pe = pl.pallas_call(
        _edge_proj_body,
        grid=(E // BE,),
        in_specs=[
            pl.BlockSpec((BE, f_edge), lambda i: (i, 0)),
            pl.BlockSpec((f_edge, D), lambda i: (0, 0)),
            pl.BlockSpec((1, D), lambda i: (0, 0)),
            pl.BlockSpec((D, H), lambda i: (0, 0)),
        ],
        out_specs=pl.BlockSpec((BE, H), lambda i: (i, 0)),
        out_shape=jax.ShapeDtypeStruct((E, H), jnp.float32),
    )(edge_attr, W_edge, b_edge.reshape(1, D), A_edge)

    z8 = jnp.zeros((NPAD, DH), jnp.float32)
    z32 = jnp.zeros((NPAD, D), jnp.float32)
    ev8, d2 = _sc_logit_pass(src, dst, t, pe, z8)
    u2 = _sc_agg_pass(src, dst, ev8, h, z32)

    out = pl.pallas_call(
        _out_proj_body,
        grid=(16,),
        in_specs=[
            pl.BlockSpec((NC, BN1, D), lambda i: (0, i, 0)),
            pl.BlockSpec((NC, BN1, DH), lambda i: (0, i, 0)),
            pl.BlockSpec((BN1, D), lambda i: (i, 0)),
            pl.BlockSpec((DH, D), lambda i: (0, 0)),
            pl.BlockSpec((D, D), lambda i: (0, 0)),
            pl.BlockSpec((1, D), lambda i: (0, 0)),
        ],
        out_specs=pl.BlockSpec((BN1, D), lambda i: (i, 0)),
        out_shape=jax.ShapeDtypeStruct((NPAD, D), jnp.float32),
    )(u2, d2, h, Rmat, W_out, b_out.reshape(1, D))
    return out[:N]


# speedup vs baseline: 65.2748x; 1.1738x over previous
import functools

import jax
import jax.numpy as jnp
from jax import lax
from jax.experimental import pallas as pl
from jax.experimental.pallas import tpu as pltpu
from jax.experimental.pallas import tpu_sc as plsc

N = 50000
E = 800000
D = 32
H = 4
DH = 8

NC = 2
NS = 16
L = 16
NW = NC * NS
EPW = E // NW
CH = 200
NCHUNK = EPW // CH
NPAD = 50048
RPT = NPAD // NS

BN = 5000
BE = 8000


def _node_proj_body(x_ref, w_ref, b_ref, asd_ref, h_ref, t_ref):
    h = jax.nn.gelu(jnp.dot(x_ref[...], w_ref[...],
                            preferred_element_type=jnp.float32) + b_ref[...])
    h_ref[...] = h
    t_ref[...] = jnp.dot(h, asd_ref[...], preferred_element_type=jnp.float32)


def _edge_proj_body(ea_ref, w_ref, b_ref, aedge_ref, pe_ref):
    e = jax.nn.gelu(jnp.dot(ea_ref[...], w_ref[...],
                            preferred_element_type=jnp.float32) + b_ref[...])
    pe_ref[...] = jnp.dot(e, aedge_ref[...], preferred_element_type=jnp.float32)


def _out_proj_body(u_ref, d_ref, h_ref, rmat_ref, w_ref, b_ref, o_ref):
    u = u_ref[0] + u_ref[1]
    dn = d_ref[0] + d_ref[1]
    rep = jnp.dot(dn, rmat_ref[...], preferred_element_type=jnp.float32)
    agg = u / (rep + 1e-20)
    o_ref[...] = jax.nn.gelu(jnp.dot(agg, w_ref[...],
                                     preferred_element_type=jnp.float32)
                             + b_ref[...]) + h_ref[...]


_SC_MESH = plsc.VectorSubcoreMesh(core_axis_name="c", subcore_axis_name="s")
_SC_PARAMS = pltpu.CompilerParams(use_tc_tiling_on_sc=False,
                                  needs_layout_passes=False)
CH1 = 1000
CH2 = 720
NCH2 = EPW // CH2
REM2 = EPW - NCH2 * CH2


@functools.partial(
    pl.kernel,
    out_type=(
        jax.ShapeDtypeStruct((E, DH), jnp.float32),
        jax.ShapeDtypeStruct((NC, NPAD, DH), jnp.float32),
    ),
    mesh=_SC_MESH,
    compiler_params=_SC_PARAMS,
    scratch_types=(
        pltpu.VMEM_SHARED((NPAD, DH), jnp.float32),
        pltpu.VMEM((CH1,), jnp.int32),
        pltpu.VMEM((CH1,), jnp.int32),
        pltpu.VMEM((CH1, DH), jnp.float32),
        pltpu.VMEM((CH1, DH), jnp.float32),
        pltpu.VMEM((CH1, DH), jnp.float32),
        pltpu.VMEM((CH1, DH), jnp.float32),
        pltpu.SemaphoreType.DMA,
    ),
)
def _sc_logit_pass(src_hbm, dst_hbm, t_hbm, pe_hbm, z8_hbm,
                   ev_out, d_out, d_sh, srcv, dstv, sg, dg, pev, evv, sem):
    cid = lax.axis_index("c")
    sid = lax.axis_index("s")
    wid = cid * NS + sid
    row0 = sid * RPT
    pltpu.sync_copy(z8_hbm.at[pl.ds(row0, RPT)], d_sh.at[pl.ds(row0, RPT)])
    pltpu.sync_copy(z8_hbm.at[pl.ds(0, CH1)], evv)
    plsc.subcore_barrier()

    iota = lax.iota(jnp.int32, L)
    row4 = iota // H
    col4 = lax.rem(iota, H)
    ebase = wid * EPW

    @pl.loop(0, EPW // CH1)
    def _chunk(j):
        off = ebase + j * CH1
        i1 = pltpu.async_copy(src_hbm.at[pl.ds(off, CH1)], srcv, sem)
        i2 = pltpu.async_copy(dst_hbm.at[pl.ds(off, CH1)], dstv, sem)
        i3 = pltpu.async_copy(pe_hbm.at[pl.ds(off, CH1)], pev, sem)
        i1.wait()
        c1 = pltpu.async_copy(t_hbm.at[srcv], sg, sem)
        i2.wait()
        c2 = pltpu.async_copy(t_hbm.at[dstv], dg, sem)
        i3.wait()
        c1.wait()
        c2.wait()

        @pl.loop(0, CH1 * H // L, unroll=4)
        def _ev(i):
            rows = row4 + i * (L // H)
            a = plsc.load_gather(sg, [rows, col4])
            b = plsc.load_gather(dg, [rows, col4 + H])
            c = plsc.load_gather(pev, [rows, col4])
            s = a + b + c
            s = jnp.maximum(s, 0.2 * s)
            plsc.store_scatter(evv, [rows, col4], jnp.exp(s))

        pltpu.sync_copy(evv, d_sh.at[dstv], add=True)
        pltpu.sync_copy(evv, ev_out.at[pl.ds(off, CH1)])

    plsc.subcore_barrier()
    pltpu.sync_copy(d_sh.at[pl.ds(row0, RPT)], d_out.at[cid, pl.ds(row0, RPT)])


@functools.partial(
    pl.kernel,
    out_type=jax.ShapeDtypeStruct((NC, NPAD, D), jnp.float32),
    mesh=_SC_MESH,
    compiler_params=_SC_PARAMS,
    scratch_types=(
        pltpu.VMEM_SHARED((NPAD, D), jnp.float32),
        pltpu.VMEM((CH2,), jnp.int32),
        pltpu.VMEM((CH2,), jnp.int32),
        pltpu.VMEM((CH2, DH), jnp.float32),
        pltpu.VMEM((CH2, D), jnp.float32),
        pltpu.SemaphoreType.DMA,
    ),
)
def _sc_agg_pass(src_hbm, dst_hbm, ev_hbm, h_hbm, z32_hbm,
                 u_out, u_sh, srcv, dstv, evv, hsv, sem):
    cid = lax.axis_index("c")
    sid = lax.axis_index("s")
    wid = cid * NS + sid
    row0 = sid * RPT
    pltpu.sync_copy(z32_hbm.at[pl.ds(row0, RPT)], u_sh.at[pl.ds(row0, RPT)])
    plsc.subcore_barrier()

    iota = lax.iota(jnp.int32, L)
    bidx = iota // DH
    ebase = wid * EPW

    def do_chunk(off, n_edges):
        i1 = pltpu.async_copy(src_hbm.at[pl.ds(off, n_edges)],
                              srcv.at[pl.ds(0, n_edges)], sem)
        i2 = pltpu.async_copy(dst_hbm.at[pl.ds(off, n_edges)],
                              dstv.at[pl.ds(0, n_edges)], sem)
        i3 = pltpu.async_copy(ev_hbm.at[pl.ds(off, n_edges)],
                              evv.at[pl.ds(0, n_edges)], sem)
        i1.wait()
        cg = pltpu.async_copy(h_hbm.at[srcv], hsv, sem)
        i2.wait()
        i3.wait()
        cg.wait()

        @pl.loop(0, CH2, unroll=8)
        def _msg(r):
            rows = jnp.full((L,), 0, jnp.int32) + r
            for half in range(2):
                cols = iota + (half * L)
                hvals = plsc.load_gather(hsv, [rows, cols])
                evb = plsc.load_gather(evv, [rows, bidx + (half * 2)])
                plsc.store_scatter(hsv, [rows, cols], hvals * evb)

        pltpu.sync_copy(hsv, u_sh.at[dstv], add=True)

    @pl.loop(0, NCH2)
    def _chunk(j):
        do_chunk(ebase + j * CH2, CH2)

    @pl.loop(REM2 // L, CH2 // L)
    def _fill(q):
        dstv[pl.ds(q * L, L)] = N + lax.rem(iota + q, jnp.int32(NPAD - N))
    do_chunk(ebase + NCH2 * CH2, REM2)

    plsc.subcore_barrier()
    pltpu.sync_copy(u_sh.at[pl.ds(row0, RPT)],
                    u_out.at[cid, pl.ds(row0, RPT)])


def kernel(x, edge_index, edge_attr, W_in, b_in, W_edge, b_edge,
           a_src, a_dst, a_edge, W_out, b_out):
    src = edge_index[0]
    dst = edge_index[1]
    f_in = x.shape[1]
    f_edge = edge_attr.shape[1]
    BN1 = NPAD // 8

    mask = (jnp.arange(D)[:, None] // DH) == jnp.arange(H)[None, :]
    A_src = jnp.where(mask, a_src.reshape(D)[:, None], 0.0).astype(jnp.float32)
    A_dst = jnp.where(mask, a_dst.reshape(D)[:, None], 0.0).astype(jnp.float32)
    A_sd = jnp.concatenate([A_src, A_dst], axis=1)
    A_edge = jnp.concatenate(
        [jnp.where(mask, a_edge.reshape(D)[:, None], 0.0).astype(jnp.float32),
         jnp.zeros((D, H), jnp.float32)], axis=1)
    Rmat = jnp.concatenate([mask.T.astype(jnp.float32),
                            jnp.zeros((H, D), jnp.float32)], axis=0)

    h, t = pl.pallas_call(
        _node_proj_body,
        grid=(8,),
        in_specs=[
            pl.BlockSpec((BN1, f_in), lambda i: (i, 0)),
            pl.BlockSpec((f_in, D), lambda i: (0, 0)),
            pl.BlockSpec((1, D), lambda i: (0, 0)),
            pl.BlockSpec((D, DH), lambda i: (0, 0)),
        ],
        out_specs=[
            pl.BlockSpec((BN1, D), lambda i: (i, 0)),
            pl.BlockSpec((BN1, DH), lambda i: (i, 0)),
        ],
        out_shape=[
            jax.ShapeDtypeStruct((NPAD, D), jnp.float32),
            jax.ShapeDtypeStruct((NPAD, DH), jnp.float32),
        ],
    )(x, W_in, b_in.reshape(1, D), A_sd)

    pe = pl.pallas_call(
        _edge_proj_body,
        grid=(E // BE,),
        in_specs=[
            pl.BlockSpec((BE, f_edge), lambda i: (i, 0)),
            pl.BlockSpec((f_edge, D), lambda i: (0, 0)),
            pl.BlockSpec((1, D), lambda i: (0, 0)),
            pl.BlockSpec((D, DH), lambda i: (0, 0)),
        ],
        out_specs=pl.BlockSpec((BE, DH), lambda i: (i, 0)),
        out_shape=jax.ShapeDtypeStruct((E, DH), jnp.float32),
    )(edge_attr, W_edge, b_edge.reshape(1, D), A_edge)

    z8 = jnp.zeros((NPAD, DH), jnp.float32)
    z32 = jnp.zeros((NPAD, D), jnp.float32)
    ev8, d2 = _sc_logit_pass(src, dst, t, pe, z8)
    u2 = _sc_agg_pass(src, dst, ev8, h, z32)

    out = pl.pallas_call(
        _out_proj_body,
        grid=(8,),
        in_specs=[
            pl.BlockSpec((NC, BN1, D), lambda i: (0, i, 0)),
            pl.BlockSpec((NC, BN1, DH), lambda i: (0, i, 0)),
            pl.BlockSpec((BN1, D), lambda i: (i, 0)),
            pl.BlockSpec((DH, D), lambda i: (0, 0)),
            pl.BlockSpec((D, D), lambda i: (0, 0)),
            pl.BlockSpec((1, D), lambda i: (0, 0)),
        ],
        out_specs=pl.BlockSpec((BN1, D), lambda i: (i, 0)),
        out_shape=jax.ShapeDtypeStruct((NPAD, D), jnp.float32),
    )(u2, d2, h, Rmat, W_out, b_out.reshape(1, D))
    return out[:N]
